# baseline (speedup 1.0000x reference)
import functools

import jax
import jax.numpy as jnp
from jax import lax
from jax.experimental import pallas as pl
from jax.experimental.pallas import tpu as pltpu
from jax.experimental.pallas import tpu_sc as plsc

BATCH = 4
SEQ = 2048
EMBED_DIM = 128

NUM_CORES = 2
NUM_SUBCORES = 16
LANES = 16
NW = NUM_CORES * NUM_SUBCORES

TOKENS = BATCH * SEQ
CHUNK = TOKENS // NW
NCH = 4
CW = CHUNK // NCH
PER_ROW = SEQ // CHUNK

_mesh = plsc.VectorSubcoreMesh(core_axis_name="c", subcore_axis_name="s")


@functools.partial(
    pl.kernel,
    mesh=_mesh,
    out_type=jax.ShapeDtypeStruct((TOKENS, EMBED_DIM), jnp.float32),
    scratch_types=[
        pltpu.VMEM((NCH, CW), jnp.int32),
        pltpu.VMEM((NCH, CW), jnp.int32),
        pltpu.VMEM((CHUNK, EMBED_DIM), jnp.float32),
        pltpu.VMEM((CHUNK, EMBED_DIM), jnp.float32),
        pltpu.SemaphoreType.DMA,
        pltpu.SemaphoreType.DMA,
        pltpu.SemaphoreType.DMA,
        pltpu.SemaphoreType.DMA,
        pltpu.SemaphoreType.DMA,
    ],
)
def _embed_sc(x_hbm, xp_hbm, tok_hbm, pos_hbm, out_hbm,
              xi, pi, tok_v, pos_v, s0, s1, s2, s3, sem_o):
    wid = lax.axis_index("s") * NUM_CORES + lax.axis_index("c")
    base = wid * CHUNK
    row = wid // PER_ROW
    col = (wid % PER_ROW) * CHUNK

    sems = (s0, s1, s2, s3)
    idx_c = []
    for k in range(NCH):
        src = pl.ds(col + k * CW, CW)
        idx_c.append(pltpu.async_copy(x_hbm.at[row, src], xi.at[k], sems[k]))
        idx_c.append(pltpu.async_copy(xp_hbm.at[row, src], pi.at[k], sems[k]))
    for c in idx_c:
        c.wait()

    tok_c = []
    pos_c = []
    for k in range(NCH):
        sl = pl.ds(k * CW, CW)
        tok_c.append(pltpu.async_copy(tok_hbm.at[xi.at[k]], tok_v.at[sl],
                                      sems[k]))
        pos_c.append(pltpu.async_copy(pos_hbm.at[pi.at[k]], pos_v.at[sl],
                                      sems[k]))

    out_c = []
    for k in range(NCH):
        tok_c[k].wait()
        pos_c[k].wait()

        def body(i, _):
            for r in range(2):
                for j in range(EMBED_DIM // LANES):
                    sl = pl.ds(j * LANES, LANES)
                    plsc.addupdate(tok_v.at[i * 2 + r, sl], pos_v[i * 2 + r, sl])
            return _

        lax.fori_loop(k * (CW // 2), (k + 1) * (CW // 2), body, 0)

        sl = pl.ds(k * CW, CW)
        out_c.append(pltpu.async_copy(
            tok_v.at[sl], out_hbm.at[pl.ds(base + k * CW, CW)], sem_o))
    for c in out_c:
        c.wait()


def kernel(x, x_pos, tok_embed, pos_embed):
    out = _embed_sc(x.astype(jnp.int32), x_pos.astype(jnp.int32),
                    tok_embed, pos_embed)
    return out.reshape(BATCH, SEQ, EMBED_DIM)


# speedup vs baseline: 1.0339x; 1.0339x over previous
import functools

import jax
import jax.numpy as jnp
from jax import lax
from jax.experimental import pallas as pl
from jax.experimental.pallas import tpu as pltpu
from jax.experimental.pallas import tpu_sc as plsc

BATCH = 4
SEQ = 2048
EMBED_DIM = 128
MAX_LENGTH = 2048

NUM_CORES = 2
NUM_SUBCORES = 16
LANES = 16
NW = NUM_CORES * NUM_SUBCORES

TOKENS = BATCH * SEQ
CHUNK = TOKENS // NW
IDX_MINOR = 128
K = CHUNK // IDX_MINOR
PER_ROW = SEQ // CHUNK

_mesh = plsc.VectorSubcoreMesh(core_axis_name="c", subcore_axis_name="s")


@functools.partial(
    pl.kernel,
    mesh=_mesh,
    out_type=jax.ShapeDtypeStruct((TOKENS, EMBED_DIM), jnp.float32),
    scratch_types=[
        pltpu.VMEM((K, IDX_MINOR), jnp.int32),
        pltpu.VMEM((K, IDX_MINOR), jnp.int32),
        pltpu.VMEM((CHUNK, EMBED_DIM), jnp.float32),
        pltpu.VMEM((CHUNK, EMBED_DIM), jnp.float32),
        pltpu.VMEM_SHARED((MAX_LENGTH, EMBED_DIM), jnp.float32),
        pltpu.SemaphoreType.DMA,
        pltpu.SemaphoreType.DMA,
        pltpu.SemaphoreType.DMA,
        pltpu.SemaphoreType.DMA,
        pltpu.SemaphoreType.DMA,
    ],
)
def _embed_sc(x_hbm, xp_hbm, tok_hbm, pos_hbm, out_hbm,
              xi, pi, tok_v, pos_v, pos_sh,
              sem_t0, sem_t1, sem_p0, sem_p1, sem_o):
    wid = lax.axis_index("s") * NUM_CORES + lax.axis_index("c")
    sid = lax.axis_index("s")
    base = wid * CHUNK
    row = wid // PER_ROW
    col = (wid % PER_ROW) * CHUNK

    sem_t = (sem_t0, sem_t1)
    sem_p = (sem_p0, sem_p1)
    idx_c = []
    for k in range(K):
        src = pl.ds(col + k * IDX_MINOR, IDX_MINOR)
        idx_c.append(pltpu.async_copy(x_hbm.at[row, src], xi.at[k], sem_t[k]))
        idx_c.append(pltpu.async_copy(xp_hbm.at[row, src], pi.at[k], sem_p[k]))

    @pl.when(sid == 0)
    def _():
        pltpu.sync_copy(pos_hbm, pos_sh)

    for c in idx_c:
        c.wait()

    tok_c = []
    for k in range(K):
        sl = pl.ds(k * IDX_MINOR, IDX_MINOR)
        tok_c.append(pltpu.async_copy(tok_hbm.at[xi.at[k]], tok_v.at[sl],
                                      sem_t[k]))

    plsc.subcore_barrier()

    pos_c = []
    for k in range(K):
        sl = pl.ds(k * IDX_MINOR, IDX_MINOR)
        pos_c.append(pltpu.async_copy(pos_sh.at[pi.at[k]], pos_v.at[sl],
                                      sem_p[k]))

    out_c = []
    for k in range(K):
        tok_c[k].wait()
        pos_c[k].wait()

        def body(i, _):
            for r in range(2):
                for j in range(EMBED_DIM // LANES):
                    sl = pl.ds(j * LANES, LANES)
                    plsc.addupdate(tok_v.at[i * 2 + r, sl], pos_v[i * 2 + r, sl])
            return _

        lax.fori_loop(k * (IDX_MINOR // 2), (k + 1) * (IDX_MINOR // 2), body, 0)

        sl = pl.ds(k * IDX_MINOR, IDX_MINOR)
        out_c.append(pltpu.async_copy(
            tok_v.at[sl], out_hbm.at[pl.ds(base + k * IDX_MINOR, IDX_MINOR)],
            sem_o))
    for c in out_c:
        c.wait()


def kernel(x, x_pos, tok_embed, pos_embed):
    out = _embed_sc(x.astype(jnp.int32), x_pos.astype(jnp.int32),
                    tok_embed, pos_embed)
    return out.reshape(BATCH, SEQ, EMBED_DIM)
